# Initial kernel scaffold; baseline (speedup 1.0000x reference)
import jax
import jax.numpy as jnp
from jax.experimental import pallas as pl


def kernel(x, edge_index, batch, W_gcn0, W_gcn1, W_fc1, W_fc2):
    raise NotImplementedError("write your pallas kernel here")


# speedup vs baseline: 9.2637x; 9.2637x over previous
import functools

import jax
import jax.numpy as jnp
from jax import lax
from jax.experimental import pallas as pl
from jax.experimental.pallas import tpu as pltpu
from jax.experimental.pallas import tpu_sc as plsc

N = 10000
NP = 10240
E = 320000
DIN = 128
H = 128
F1 = 64
G = 64
NW = 32
C = 80
K = 128
EP = NW * C * K
RPT = NP // 16
BLK = 1024
NBLK = NP // BLK

_mesh = plsc.VectorSubcoreMesh(core_axis_name="c", subcore_axis_name="s",
                               num_cores=2, num_subcores=16)


def _leaky(v):
    return jnp.where(v >= 0, v, 0.01 * v)


@functools.partial(
    pl.kernel,
    out_type=jax.ShapeDtypeStruct((2, NP, H), jnp.float32),
    mesh=_mesh,
    scratch_types=[
        pltpu.VMEM((C, K), jnp.int32),
        pltpu.VMEM((K, H), jnp.float32),
        pltpu.VMEM_SHARED((NP, H), jnp.float32),
    ],
)
def _deg_kernel(dst_hbm, zeros_hbm, out_hbm, dst_v, ones_v, acc_sh):
    cid = lax.axis_index("c")
    sid = lax.axis_index("s")
    wid = sid * 2 + cid
    pltpu.sync_copy(dst_hbm.at[wid], dst_v)
    ones16 = jnp.ones((16,), jnp.float32)

    def fill(i, carry):
        def inner(j, carry2):
            ones_v[i, pl.ds(j * 16, 16)] = ones16
            return carry2
        return lax.fori_loop(0, H // 16, inner, carry)

    lax.fori_loop(0, K, fill, 0)
    r0 = sid * RPT
    pltpu.sync_copy(zeros_hbm.at[pl.ds(r0, RPT)], acc_sh.at[pl.ds(r0, RPT)])
    plsc.subcore_barrier()

    def body(i, carry):
        pltpu.sync_copy(ones_v, acc_sh.at[dst_v.at[i]], add=True)
        return carry

    lax.fori_loop(0, C, body, 0)
    plsc.subcore_barrier()
    pltpu.sync_copy(acc_sh.at[pl.ds(r0, RPT)], out_hbm.at[cid, pl.ds(r0, RPT)])


@functools.partial(
    pl.kernel,
    out_type=jax.ShapeDtypeStruct((2, NP, H), jnp.float32),
    mesh=_mesh,
    scratch_types=[
        pltpu.VMEM((C, K), jnp.int32),
        pltpu.VMEM((C, K), jnp.int32),
        pltpu.VMEM((K, H), jnp.float32),
        pltpu.VMEM_SHARED((NP, H), jnp.float32),
        pltpu.SemaphoreType.DMA,
    ],
)
def _edge_kernel(z_hbm, zeros_hbm, src_hbm, dst_hbm, out_hbm,
                 src_v, dst_v, rows_v, acc_sh, sem):
    cid = lax.axis_index("c")
    sid = lax.axis_index("s")
    wid = sid * 2 + cid
    pltpu.sync_copy(src_hbm.at[wid], src_v)
    pltpu.sync_copy(dst_hbm.at[wid], dst_v)
    r0 = sid * RPT
    pltpu.sync_copy(zeros_hbm.at[pl.ds(r0, RPT)], acc_sh.at[pl.ds(r0, RPT)])
    plsc.subcore_barrier()

    def body(i, carry):
        pltpu.async_copy(z_hbm.at[src_v.at[i]], rows_v, sem).wait()
        pltpu.sync_copy(rows_v, acc_sh.at[dst_v.at[i]], add=True)
        return carry

    lax.fori_loop(0, C, body, 0)
    plsc.subcore_barrier()
    pltpu.sync_copy(acc_sh.at[pl.ds(r0, RPT)], out_hbm.at[cid, pl.ds(r0, RPT)])


def _tc1_body(x_ref, w_ref, degp_ref, z0_ref, dis_ref):
    deg = (degp_ref[0] + degp_ref[1])[:, 0:1] + 1.0
    dis = lax.rsqrt(deg)
    xw = jnp.dot(x_ref[...], w_ref[...],
                 preferred_element_type=jnp.float32)
    z0_ref[...] = dis * xw
    dis_ref[...] = dis


_tc1 = pl.pallas_call(
    _tc1_body,
    grid=(NBLK,),
    in_specs=[
        pl.BlockSpec((BLK, DIN), lambda i: (i, 0)),
        pl.BlockSpec((DIN, H), lambda i: (0, 0)),
        pl.BlockSpec((2, BLK, H), lambda i: (0, i, 0)),
    ],
    out_specs=[
        pl.BlockSpec((BLK, H), lambda i: (i, 0)),
        pl.BlockSpec((BLK, 1), lambda i: (i, 0)),
    ],
    out_shape=[
        jax.ShapeDtypeStruct((NP, H), jnp.float32),
        jax.ShapeDtypeStruct((NP, 1), jnp.float32),
    ],
)


def _tc2_body(p_ref, z0_ref, dis_ref, w_ref, z1_ref):
    s = p_ref[0] + p_ref[1] + z0_ref[...]
    h = _leaky(dis_ref[...] * s)
    hw = jnp.dot(h, w_ref[...],
                 preferred_element_type=jnp.float32)
    z1_ref[...] = dis_ref[...] * hw


_tc2 = pl.pallas_call(
    _tc2_body,
    grid=(NBLK,),
    in_specs=[
        pl.BlockSpec((2, BLK, H), lambda i: (0, i, 0)),
        pl.BlockSpec((BLK, H), lambda i: (i, 0)),
        pl.BlockSpec((BLK, 1), lambda i: (i, 0)),
        pl.BlockSpec((H, H), lambda i: (0, 0)),
    ],
    out_specs=pl.BlockSpec((BLK, H), lambda i: (i, 0)),
    out_shape=jax.ShapeDtypeStruct((NP, H), jnp.float32),
)


def _tc3_body(p_ref, z1_ref, dis_ref, wf1_ref, wf2_ref, batch_ref,
              out_ref, sums, cnts):
    i = pl.program_id(0)
    s = p_ref[0] + p_ref[1] + z1_ref[...]
    h = _leaky(dis_ref[...] * s)
    y1 = _leaky(jnp.dot(h, wf1_ref[...],
                        preferred_element_type=jnp.float32))
    y2 = _leaky(jnp.dot(y1, wf2_ref[...],
                        preferred_element_type=jnp.float32))
    gi = lax.broadcasted_iota(jnp.int32, (BLK, G), 1)
    mask = (batch_ref[...] == gi).astype(jnp.float32)
    dims = (((0,), (0,)), ((), ()))
    ps = lax.dot_general(mask, y2, dims,
                         preferred_element_type=jnp.float32)
    pc = lax.dot_general(mask, jnp.ones((BLK, 1), jnp.float32), dims,
                         preferred_element_type=jnp.float32)

    @pl.when(i == 0)
    def _():
        sums[...] = jnp.zeros((G, 1), jnp.float32)
        cnts[...] = jnp.zeros((G, 1), jnp.float32)

    sums[...] += ps
    cnts[...] += pc

    @pl.when(i == NBLK - 1)
    def _():
        out_ref[...] = sums[...] / jnp.maximum(cnts[...], 1.0)


_tc3 = pl.pallas_call(
    _tc3_body,
    grid=(NBLK,),
    in_specs=[
        pl.BlockSpec((2, BLK, H), lambda i: (0, i, 0)),
        pl.BlockSpec((BLK, H), lambda i: (i, 0)),
        pl.BlockSpec((BLK, 1), lambda i: (i, 0)),
        pl.BlockSpec((H, F1), lambda i: (0, 0)),
        pl.BlockSpec((F1, 1), lambda i: (0, 0)),
        pl.BlockSpec((BLK, 1), lambda i: (i, 0)),
    ],
    out_specs=pl.BlockSpec((G, 1), lambda i: (0, 0)),
    out_shape=jax.ShapeDtypeStruct((G, 1), jnp.float32),
    scratch_shapes=[
        pltpu.VMEM((G, 1), jnp.float32),
        pltpu.VMEM((G, 1), jnp.float32),
    ],
)


def kernel(x, edge_index, batch, W_gcn0, W_gcn1, W_fc1, W_fc2):
    x_pad = jnp.zeros((NP, DIN), jnp.float32).at[:N].set(x)
    pad = jnp.full((EP - E,), N, jnp.int32)
    src_p = jnp.concatenate([edge_index[0], pad]).reshape(NW, C, K)
    dst_p = jnp.concatenate([edge_index[1], pad]).reshape(NW, C, K)
    batch_p = jnp.concatenate(
        [batch, jnp.full((NP - N,), G, jnp.int32)]).reshape(NP, 1)
    zeros_np = jnp.zeros((NP, H), jnp.float32)
    degp = _deg_kernel(dst_p, zeros_np)
    z0, dis = _tc1(x_pad, W_gcn0, degp)
    p1 = _edge_kernel(z0, zeros_np, src_p, dst_p)
    z1 = _tc2(p1, z0, dis, W_gcn1)
    p2 = _edge_kernel(z1, zeros_np, src_p, dst_p)
    return _tc3(p2, z1, dis, W_fc1, W_fc2, batch_p)
